# Initial kernel scaffold; baseline (speedup 1.0000x reference)
import jax
import jax.numpy as jnp
from jax.experimental import pallas as pl


def kernel(x, embed_table, W, b):
    raise NotImplementedError("write your pallas kernel here")


# speedup vs baseline: 1.9963x; 1.9963x over previous
import functools

import jax
import jax.numpy as jnp
from jax import lax
from jax.experimental import pallas as pl
from jax.experimental.pallas import tpu as pltpu
from jax.experimental.pallas import tpu_sc as plsc

LANES = 16


def _sc_pool(x, embed_table):
    B, L = x.shape
    _, D2 = embed_table.shape
    NF = D2 // LANES

    info = plsc.get_sparse_core_info()
    NC, NS = info.num_cores, info.num_subcores
    NW = NC * NS
    b_per_w = B // NW

    mesh = plsc.VectorSubcoreMesh(core_axis_name="c", subcore_axis_name="s")
    inv_l = 1.0 / L
    CHUNK = 64
    n_chunks = b_per_w // CHUNK

    @functools.partial(
        pl.kernel,
        mesh=mesh,
        out_type=jax.ShapeDtypeStruct((B, D2), jnp.float32),
        scratch_types=[
            pltpu.VMEM((CHUNK, L), jnp.int32),
            pltpu.VMEM((L, D2), jnp.float32),
            pltpu.VMEM((CHUNK, D2), jnp.float32),
            pltpu.SemaphoreType.DMA,
        ],
    )
    def k(x_hbm, table_hbm, out_hbm, idx_v, rows_v, pooled_v, sem):
        wid = lax.axis_index("s") * NC + lax.axis_index("c")
        base = wid * b_per_w

        def chunk_body(c, carry):
            cbase = base + c * CHUNK
            pltpu.sync_copy(x_hbm.at[pl.ds(cbase, CHUNK)], idx_v)

            def row(i, carry2):
                pltpu.async_copy(table_hbm.at[idx_v.at[i]], rows_v, sem).wait()

                def acc_body(j, accs):
                    return tuple(
                        accs[f] + rows_v[j, pl.ds(f * LANES, LANES)]
                        for f in range(NF)
                    )

                accs = lax.fori_loop(
                    0, L, acc_body,
                    tuple(jnp.zeros((LANES,), jnp.float32) for _ in range(NF)),
                )
                for f in range(NF):
                    pooled_v[i, pl.ds(f * LANES, LANES)] = accs[f] * inv_l
                return carry2

            lax.fori_loop(0, CHUNK, row, 0)
            pltpu.sync_copy(pooled_v, out_hbm.at[pl.ds(cbase, CHUNK)])
            return carry

        lax.fori_loop(0, n_chunks, chunk_body, 0)

    return k(x, embed_table)


def _tc_head(pooled, w_t, bias):
    B, D2 = pooled.shape
    D = w_t.shape[1]
    BM = 2048

    def body(p_ref, w_ref, b_ref, o_ref):
        acc = jnp.dot(p_ref[...], w_ref[...], preferred_element_type=jnp.float32)
        o_ref[...] = jnp.tanh(acc + b_ref[...])

    return pl.pallas_call(
        body,
        grid=(B // BM,),
        in_specs=[
            pl.BlockSpec((BM, D2), lambda i: (i, 0)),
            pl.BlockSpec((D2, D), lambda i: (0, 0)),
            pl.BlockSpec((1, D), lambda i: (0, 0)),
        ],
        out_specs=pl.BlockSpec((BM, D), lambda i: (i, 0)),
        out_shape=jax.ShapeDtypeStruct((B, D), jnp.float32),
    )(pooled, w_t, bias.reshape(1, D))


def kernel(x, embed_table, W, b):
    x = x.astype(jnp.int32)
    pooled = _sc_pool(x, embed_table)
    return _tc_head(pooled, W.T, b)


# speedup vs baseline: 3.2842x; 1.6451x over previous
import functools

import jax
import jax.numpy as jnp
from jax import lax
from jax.experimental import pallas as pl
from jax.experimental.pallas import tpu as pltpu
from jax.experimental.pallas import tpu_sc as plsc

LANES = 16


def _sc_pool(x, embed_table):
    B, L = x.shape
    _, D2 = embed_table.shape
    NF = D2 // LANES

    info = plsc.get_sparse_core_info()
    NC, NS = info.num_cores, info.num_subcores
    NW = NC * NS
    b_per_w = B // NW

    mesh = plsc.VectorSubcoreMesh(core_axis_name="c", subcore_axis_name="s")
    inv_l = 1.0 / L
    CHUNK = 64
    NBUF = 4
    n_steps = b_per_w // NBUF

    @functools.partial(
        pl.kernel,
        mesh=mesh,
        out_type=jax.ShapeDtypeStruct((B, D2), jnp.float32),
        scratch_types=[
            pltpu.VMEM((b_per_w, L), jnp.int32),
            pltpu.VMEM((NBUF, L, D2), jnp.float32),
            pltpu.VMEM((CHUNK, D2), jnp.float32),
            pltpu.SemaphoreType.DMA,
            pltpu.SemaphoreType.DMA,
            pltpu.SemaphoreType.DMA,
            pltpu.SemaphoreType.DMA,
        ],
    )
    def k(x_hbm, table_hbm, dummy_hbm, out_hbm, idx_v, rows_v, pooled_v,
          s0, s1, s2, s3):
        sems = (s0, s1, s2, s3)
        wid = lax.axis_index("s") * NC + lax.axis_index("c")
        base = wid * b_per_w

        pltpu.sync_copy(x_hbm.at[pl.ds(base, b_per_w)], idx_v)

        for p in range(NBUF):
            pltpu.async_copy(table_hbm.at[idx_v.at[p]], rows_v.at[p], sems[p])

        def step(i2, carry):
            for p in range(NBUF):
                i = i2 * NBUF + p
                pltpu.make_async_copy(dummy_hbm, rows_v.at[p], sems[p]).wait()
                islot = lax.rem(i, CHUNK)
                FB = 2
                for fg in range(0, NF, FB):
                    accs = [rows_v[p, 0, pl.ds((fg + f) * LANES, LANES)]
                            for f in range(FB)]
                    for j in range(1, L):
                        for f in range(FB):
                            accs[f] = accs[f] + rows_v[
                                p, j, pl.ds((fg + f) * LANES, LANES)]
                    for f in range(FB):
                        pooled_v[islot, pl.ds((fg + f) * LANES, LANES)] = (
                            accs[f] * inv_l)
                @pl.when(i2 < n_steps - 1)
                def _():
                    pltpu.async_copy(
                        table_hbm.at[idx_v.at[i + NBUF]], rows_v.at[p], sems[p]
                    )
                if p == NBUF - 1:
                    @pl.when(lax.rem(i2, CHUNK // NBUF) == CHUNK // NBUF - 1)
                    def _():
                        cstart = pl.multiple_of(base + i - (CHUNK - 1), CHUNK)
                        pltpu.sync_copy(
                            pooled_v, out_hbm.at[pl.ds(cstart, CHUNK)]
                        )
            return carry

        lax.fori_loop(0, n_steps, step, 0)

    return k(x, embed_table, jnp.zeros((L, D2), jnp.float32))


def _tc_head(pooled, w_t, bias):
    B, D2 = pooled.shape
    D = w_t.shape[1]
    BM = 2048

    def body(p_ref, w_ref, b_ref, o_ref):
        acc = jnp.dot(p_ref[...], w_ref[...], preferred_element_type=jnp.float32)
        o_ref[...] = jnp.tanh(acc + b_ref[...])

    return pl.pallas_call(
        body,
        grid=(B // BM,),
        in_specs=[
            pl.BlockSpec((BM, D2), lambda i: (i, 0)),
            pl.BlockSpec((D2, D), lambda i: (0, 0)),
            pl.BlockSpec((1, D), lambda i: (0, 0)),
        ],
        out_specs=pl.BlockSpec((BM, D), lambda i: (i, 0)),
        out_shape=jax.ShapeDtypeStruct((B, D), jnp.float32),
    )(pooled, w_t, bias.reshape(1, D))


def kernel(x, embed_table, W, b):
    x = x.astype(jnp.int32)
    pooled = _sc_pool(x, embed_table)
    return _tc_head(pooled, W.T, b)
